# baseline (speedup 1.0000x reference)
import functools

import jax
import jax.numpy as jnp
from jax import lax
from jax.experimental import pallas as pl
from jax.experimental.pallas import tpu as pltpu
from jax.experimental.pallas import tpu_sc as plsc

B = 16384
N_NUM = 13
N_CAT = 26
VOCAB = 100000
D = 64
N_TOK = 1 + N_NUM + N_CAT
NROWS = N_TOK * D
DENSE_ROWS = (1 + N_NUM) * D
CAT_ROWS = N_CAT * D
LANES = 16

NC = 2
NS = 16
NW = NC * NS
DENSE_PER_W = DENSE_ROWS // NW
CAT_PER_W = CAT_ROWS // NW
BC = 2048
NBC = B // BC
NVC = BC // LANES
BCL = 4096
NBCL = B // BCL
NVCL = BCL // LANES


def _splat_gather(ref, pos):
    return plsc.load_gather(ref, [jnp.full((LANES,), pos, jnp.int32)])


def _tokenizer_body(xnumt_hbm, xcatt_hbm, w_hbm, bias_hbm, cls_hbm, tbl_hbm,
                    out_hbm, row_v, idxcol_v, resl_v0, resl_v1,
                    xch_v0, xch_v1, w_v, bias_v, cls_v, osem, rsem0, rsem1,
                    xsem):
    wid = lax.axis_index("s") * NC + lax.axis_index("c")

    pltpu.sync_copy(w_hbm, w_v)
    pltpu.sync_copy(bias_hbm, bias_v)
    pltpu.sync_copy(cls_hbm, cls_v)
    resl = [resl_v0, resl_v1]
    res = [resl_v0.at[pl.ds(0, BC)], resl_v1.at[pl.ds(0, BC)]]
    xch = [xch_v0, xch_v1]
    HV = 50048
    HV2 = VOCAB - HV

    def emit_row(row, make_chunk):
        handles = [None, None]
        for c in range(NBC):
            r = res[c % 2]
            if handles[c % 2] is not None:
                handles[c % 2].wait()
            make_chunk(c, r)
            handles[c % 2] = pltpu.async_copy(
                r, out_hbm.at[row, pl.ds(c * BC, BC)], osem)
        for h in handles:
            h.wait()

    def emit_row_big(row, make_chunk):
        handles = [None, None]
        for c in range(NBCL):
            r = resl[c % 2]
            if handles[c % 2] is not None:
                handles[c % 2].wait()
            make_chunk(c, r)
            handles[c % 2] = pltpu.async_copy(
                r, out_hbm.at[row, pl.ds(c * BCL, BCL)], osem)
        for h in handles:
            h.wait()

    def dense_body(j, carry):
        row = wid + NW * j

        def cls_chunk(c, r):
            cv = _splat_gather(cls_v, row)

            def vb(k, carry2):
                r[pl.ds(LANES * k, LANES)] = cv
                return carry2
            lax.fori_loop(0, NVC, vb, 0, unroll=4)

        def num_chunk(c, r):
            q = row - D
            i = q // D
            wv = _splat_gather(w_v, q)
            bv = _splat_gather(bias_v, q)
            xc = xch[c % 2]
            pltpu.make_async_copy(
                xnumt_hbm.at[i, pl.ds(c * BC, BC)], xc, xsem).wait()
            if c + 1 < NBC:
                pltpu.async_copy(
                    xnumt_hbm.at[i, pl.ds((c + 1) * BC, BC)],
                    xch[(c + 1) % 2], xsem)

            def vb(k, carry2):
                r[pl.ds(LANES * k, LANES)] = (
                    xc[pl.ds(LANES * k, LANES)] * wv + bv)
                return carry2
            lax.fori_loop(0, NVC, vb, 0, unroll=8)

        def do_cls(_):
            emit_row(row, cls_chunk)
            return 0

        def do_num(_):
            i = (row - D) // D
            pltpu.async_copy(xnumt_hbm.at[i, pl.ds(0, BC)], xch[0], xsem)
            emit_row(row, num_chunk)
            return 0

        lax.cond(row < D, do_cls, do_num, 0)
        return carry

    lax.fori_loop(0, DENSE_PER_W, dense_body, 0)

    def cat_body(j, prev_f):
        q = wid * CAT_PER_W + j
        f = q // D
        d = q - f * D
        row = DENSE_ROWS + q

        pltpu.async_copy(tbl_hbm.at[f, d], row_v, rsem0)

        def load_idx(_):
            pltpu.sync_copy(xcatt_hbm.at[f], idxcol_v)
            return f

        prev_f = lax.cond(f != prev_f, load_idx, lambda _: prev_f, 0)

        pltpu.make_async_copy(tbl_hbm.at[f, d], row_v, rsem0).wait()

        def cat_chunk(c, r):
            def vb(k, carry2):
                iv = idxcol_v[pl.ds(c * BCL + LANES * k, LANES)]
                r[pl.ds(LANES * k, LANES)] = plsc.load_gather(row_v, [iv])
                return carry2
            lax.fori_loop(0, NVCL, vb, 0, unroll=8)

        emit_row_big(row, cat_chunk)
        return prev_f

    lax.fori_loop(0, CAT_PER_W, cat_body, jnp.int32(-1))


@jax.jit
def _tokenizer(xnumt, xcatt, w_flat, bias_flat, cls_flat, tbl_t):
    mesh = plsc.VectorSubcoreMesh(core_axis_name="c", subcore_axis_name="s")
    kern = pl.kernel(
        _tokenizer_body,
        out_type=jax.ShapeDtypeStruct((NROWS, B), jnp.float32),
        mesh=mesh,
        scratch_types=[
            pltpu.VMEM((VOCAB,), jnp.float32),
            pltpu.VMEM((B,), jnp.int32),
            pltpu.VMEM((BCL,), jnp.float32),
            pltpu.VMEM((BCL,), jnp.float32),
            pltpu.VMEM((BC,), jnp.float32),
            pltpu.VMEM((BC,), jnp.float32),
            pltpu.VMEM((N_NUM * D,), jnp.float32),
            pltpu.VMEM((N_NUM * D,), jnp.float32),
            pltpu.VMEM((D,), jnp.float32),
            pltpu.SemaphoreType.DMA,
            pltpu.SemaphoreType.DMA,
            pltpu.SemaphoreType.DMA,
            pltpu.SemaphoreType.DMA,
        ],
        compiler_params=pltpu.CompilerParams(
            use_tc_tiling_on_sc=True, needs_layout_passes=False),
    )
    return kern(xnumt, xcatt, w_flat, bias_flat, cls_flat, tbl_t)


def kernel(x_num, x_cat, weight, bias, cls_token, tables):
    out = _tokenizer(
        x_num.T,
        x_cat.astype(jnp.int32).T,
        weight.reshape(N_NUM * D),
        bias.reshape(N_NUM * D),
        cls_token.reshape(D),
        tables.transpose(0, 2, 1),
    )
    return out.reshape(N_TOK, D, B).transpose(2, 0, 1)


# speedup vs baseline: 1.7620x; 1.7620x over previous
import functools

import jax
import jax.numpy as jnp
from jax import lax
from jax.experimental import pallas as pl
from jax.experimental.pallas import tpu as pltpu
from jax.experimental.pallas import tpu_sc as plsc

B = 16384
N_NUM = 13
N_CAT = 26
VOCAB = 100000
D = 64
N_TOK = 1 + N_NUM + N_CAT
NROWS = N_TOK * D
DENSE_ROWS = (1 + N_NUM) * D
CAT_ROWS = N_CAT * D
LANES = 16

NC = 2
NS = 16
NW = NC * NS
DENSE_PER_W = DENSE_ROWS // NW
CAT_PER_W = CAT_ROWS // NW
BC = 2048
NBC = B // BC
NVC = BC // LANES
BCL = 4096
NBCL = B // BCL
NVCL = BCL // LANES


def _splat_gather(ref, pos):
    return plsc.load_gather(ref, [jnp.full((LANES,), pos, jnp.int32)])


def _tokenizer_body(xnumt_hbm, xcatt_hbm, w_hbm, bias_hbm, cls_hbm, tbl_hbm,
                    out_hbm, row_v, idxcol_v, resl_v0, resl_v1,
                    xch_v0, xch_v1, w_v, bias_v, cls_v, osem, rsem0, rsem1,
                    xsem):
    wid = lax.axis_index("s") * NC + lax.axis_index("c")

    pltpu.sync_copy(w_hbm, w_v)
    pltpu.sync_copy(bias_hbm, bias_v)
    pltpu.sync_copy(cls_hbm, cls_v)
    resl = [resl_v0, resl_v1]
    res = [resl_v0.at[pl.ds(0, BC)], resl_v1.at[pl.ds(0, BC)]]
    xch = [xch_v0, xch_v1]
    HV = 50048
    HV2 = VOCAB - HV

    def emit_row(row, make_chunk):
        handles = [None, None]
        for c in range(NBC):
            r = res[c % 2]
            if handles[c % 2] is not None:
                handles[c % 2].wait()
            make_chunk(c, r)
            handles[c % 2] = pltpu.async_copy(
                r, out_hbm.at[row, pl.ds(c * BC, BC)], osem)
        for h in handles:
            h.wait()

    def emit_row_big(row, make_chunk):
        handles = [None, None]
        for c in range(NBCL):
            r = resl[c % 2]
            if handles[c % 2] is not None:
                handles[c % 2].wait()
            make_chunk(c, r)
            handles[c % 2] = pltpu.async_copy(
                r, out_hbm.at[row, pl.ds(c * BCL, BCL)], osem)
        for h in handles:
            h.wait()

    def dense_body(j, carry):
        row = wid + NW * j

        def cls_chunk(c, r):
            cv = _splat_gather(cls_v, row)

            @plsc.parallel_loop(0, NVC, unroll=8)
            def vb(k):
                r[pl.ds(LANES * k, LANES)] = cv

        def num_chunk(c, r):
            q = row - D
            i = q // D
            wv = _splat_gather(w_v, q)
            bv = _splat_gather(bias_v, q)
            xc = xch[c % 2]
            pltpu.make_async_copy(
                xnumt_hbm.at[i, pl.ds(c * BC, BC)], xc, xsem).wait()
            if c + 1 < NBC:
                pltpu.async_copy(
                    xnumt_hbm.at[i, pl.ds((c + 1) * BC, BC)],
                    xch[(c + 1) % 2], xsem)

            @plsc.parallel_loop(0, NVC, unroll=8)
            def vb(k):
                r[pl.ds(LANES * k, LANES)] = (
                    xc[pl.ds(LANES * k, LANES)] * wv + bv)

        def do_cls(_):
            emit_row(row, cls_chunk)
            return 0

        def do_num(_):
            i = (row - D) // D
            pltpu.async_copy(xnumt_hbm.at[i, pl.ds(0, BC)], xch[0], xsem)
            emit_row(row, num_chunk)
            return 0

        lax.cond(row < D, do_cls, do_num, 0)
        return carry

    lax.fori_loop(0, DENSE_PER_W, dense_body, 0)

    def cat_body(j, prev_f):
        q = wid * CAT_PER_W + j
        f = q // D
        d = q - f * D
        row = DENSE_ROWS + q

        pltpu.async_copy(tbl_hbm.at[f, d], row_v, rsem0)

        def load_idx(_):
            pltpu.sync_copy(xcatt_hbm.at[f], idxcol_v)
            return f

        prev_f = lax.cond(f != prev_f, load_idx, lambda _: prev_f, 0)

        pltpu.make_async_copy(tbl_hbm.at[f, d], row_v, rsem0).wait()

        def cat_chunk(c, r):
            @plsc.parallel_loop(0, NVCL, unroll=8)
            def vb(k):
                iv = idxcol_v[pl.ds(c * BCL + LANES * k, LANES)]
                r[pl.ds(LANES * k, LANES)] = plsc.load_gather(row_v, [iv])

        emit_row_big(row, cat_chunk)
        return prev_f

    lax.fori_loop(0, CAT_PER_W, cat_body, jnp.int32(-1))


@jax.jit
def _tokenizer(xnumt, xcatt, w_flat, bias_flat, cls_flat, tbl_t):
    mesh = plsc.VectorSubcoreMesh(core_axis_name="c", subcore_axis_name="s")
    kern = pl.kernel(
        _tokenizer_body,
        out_type=jax.ShapeDtypeStruct((NROWS, B), jnp.float32),
        mesh=mesh,
        scratch_types=[
            pltpu.VMEM((VOCAB,), jnp.float32),
            pltpu.VMEM((B,), jnp.int32),
            pltpu.VMEM((BCL,), jnp.float32),
            pltpu.VMEM((BCL,), jnp.float32),
            pltpu.VMEM((BC,), jnp.float32),
            pltpu.VMEM((BC,), jnp.float32),
            pltpu.VMEM((N_NUM * D,), jnp.float32),
            pltpu.VMEM((N_NUM * D,), jnp.float32),
            pltpu.VMEM((D,), jnp.float32),
            pltpu.SemaphoreType.DMA,
            pltpu.SemaphoreType.DMA,
            pltpu.SemaphoreType.DMA,
            pltpu.SemaphoreType.DMA,
        ],
        compiler_params=pltpu.CompilerParams(
            use_tc_tiling_on_sc=True, needs_layout_passes=False),
    )
    return kern(xnumt, xcatt, w_flat, bias_flat, cls_flat, tbl_t)


def kernel(x_num, x_cat, weight, bias, cls_token, tables):
    out = _tokenizer(
        x_num.T,
        x_cat.astype(jnp.int32).T,
        weight.reshape(N_NUM * D),
        bias.reshape(N_NUM * D),
        cls_token.reshape(D),
        tables.transpose(0, 2, 1),
    )
    return out.reshape(N_TOK, D, B).transpose(2, 0, 1)
